# baseline (speedup 1.0000x reference)
import jax
import jax.numpy as jnp
from jax import lax
from jax.experimental import pallas as pl
from jax.experimental.pallas import tpu as pltpu

N = 16384
C = 1000
RB = 2048
GRID = N // RB


def _tc_body(tgt_ref, rew_ref, prob_ref, par_ref, out_ml, out_ls, acc_ref):
    i = pl.program_id(0)

    @pl.when(i == 0)
    def _():
        acc_ref[0] = 0.0

    pb = prob_ref[...]
    t2 = tgt_ref[...]
    r2 = rew_ref[...]
    colid = lax.broadcasted_iota(jnp.int32, (RB, C), 1)
    part = jnp.sum(jnp.where(colid == t2, pb * r2, 0.0))
    acc_ref[0] += part

    @pl.when(i == GRID - 1)
    def _():
        loss = -acc_ref[0]
        c1 = par_ref[0]
        c2 = par_ref[1]
        c3 = par_ref[2]
        l2 = par_ref[3]
        l3 = par_ref[4]
        z = jnp.zeros((8, 128), jnp.float32)
        ml = (jnp.exp(z - c1) * (loss * loss) + c1 +
              jnp.exp(z - c2) * (l2 * l2) + c2 +
              jnp.exp(z - c3) * (l3 * l3) + c3)
        out_ml[...] = ml
        out_ls[...] = z + loss


@jax.jit
def _ganloss_tc(tgt2, rew2, prob, params):
    return pl.pallas_call(
        _tc_body,
        grid=(GRID,),
        in_specs=[
            pl.BlockSpec((RB, 1), lambda i: (i, 0)),
            pl.BlockSpec((RB, 1), lambda i: (i, 0)),
            pl.BlockSpec((RB, C), lambda i: (i, 0)),
            pl.BlockSpec(memory_space=pltpu.SMEM),
        ],
        out_specs=[
            pl.BlockSpec((8, 128), lambda i: (0, 0)),
            pl.BlockSpec((8, 128), lambda i: (0, 0)),
        ],
        out_shape=[
            jax.ShapeDtypeStruct((8, 128), jnp.float32),
            jax.ShapeDtypeStruct((8, 128), jnp.float32),
        ],
        scratch_shapes=[pltpu.SMEM((1,), jnp.float32)],
        compiler_params=pltpu.CompilerParams(
            dimension_semantics=("arbitrary",)),
    )(tgt2, rew2, prob, params)


def kernel(prob, target, reward, _loss2, _loss3, cf1, cf2, cf3):
    tgt2 = target.astype(jnp.int32).reshape(N, 1)
    rew2 = reward.reshape(N, 1)
    params = jnp.concatenate(
        [cf1, cf2, cf3, _loss2, _loss3,
         jnp.zeros((3,), jnp.float32)]).astype(jnp.float32)
    ml, ls = _ganloss_tc(tgt2, rew2, prob, params)
    return (ml[0, 0], ls[0, 0])
